# baseline (speedup 1.0000x reference)
import functools

import jax
import jax.numpy as jnp
from jax import lax
from jax.experimental import pallas as pl
from jax.experimental.pallas import tpu as pltpu
from jax.experimental.pallas import tpu_sc as plsc

B, J, I, N = 16, 32, 2048, 16
L = 16
NC, NS = 2, 16
IH = I // 2
SW = IH * N
XROW = 8 * 128
NEG = float("-inf")
POS = float("inf")


def _sort16(v):
    k = 2
    while k <= 16:
        step = k // 2
        while step >= 1:
            for i in range(16):
                m = i ^ step
                if m > i:
                    a, b = v[i], v[m]
                    mn = jnp.minimum(a, b)
                    mx = jnp.maximum(a, b)
                    if (i & k) == 0:
                        v[i], v[m] = mn, mx
                    else:
                        v[i], v[m] = mx, mn
            step //= 2
        k *= 2
    return v


def _tree_max(vs):
    while len(vs) > 1:
        vs = [jnp.maximum(vs[i], vs[i + 1]) for i in range(0, len(vs) - 1, 2)] + (
            [vs[-1]] if len(vs) % 2 else [])
    return vs[0]


def _rsqrt_vec(x):
    i = plsc.bitcast(x, jnp.int32)
    y = plsc.bitcast(jnp.int32(0x5F3759DF) - (i >> 1), jnp.float32)
    for _ in range(3):
        y = y * (1.5 - 0.5 * x * y * y)
    return y


def _routing_body(u_hbm, scale_hbm, v_out, u_buf, bbuf, cbuf, wbuf,
                  svec, pbuf, vvec, scvec, idxt, fbuf, shared, dsem):
    c_ax = lax.axis_index("c")
    s_ax = lax.axis_index("s")
    b_idx = c_ax * 8 + lax.div(s_ax, 2)
    ihalf = lax.rem(s_ax, 2)
    x00 = b_idx * (J * I * N // XROW) + ihalf * (SW // XROW)

    pltpu.sync_copy(scale_hbm, scvec)
    scale = scvec[...]
    zero = jnp.zeros((L,), jnp.float32)
    minf = jnp.full((L,), NEG, jnp.float32)
    pinf = jnp.full((L,), POS, jnp.float32)
    iota16 = lax.iota(jnp.int32, L)

    for k in range(4):
        for n in range(N):
            w = iota16 * N + (16 * N * k + n)
            idxt[16 * k + n, :] = w >> 7
            idxt[64 + 16 * k + n, :] = w & 127

    def start_slab(j, p):
        src = u_hbm.at[pl.ds(x00 + j * (I * N // XROW), SW // XROW)]
        return pltpu.make_async_copy(src, u_buf.at[p], dsem.at[p])

    def u_nat(p, q, r):
        return u_buf[p, q, r // 128, pl.ds(r % 128, L)]

    def zero_svec(j, carry):
        svec[j, :] = zero
        return carry

    def p0_slab(j, carry):
        p = lax.rem(j, 2)
        start_slab(j, p).wait()
        pl.when(j + 1 < J)(lambda: start_slab(j + 1, 1 - p).start())

        def blk(q, acc2):
            accs = [zero] * 4
            for k in range(4):
                for t in range(16):
                    accs[k] = accs[k] + u_nat(p, q, (16 * k + t) * N)
            return acc2 + ((accs[0] + accs[1]) + (accs[2] + accs[3]))
        acc = lax.fori_loop(0, IH // 64, blk, zero)
        svec[j, :] = acc
        return carry

    start_slab(0, 0).start()
    lax.fori_loop(0, J, p0_slab, 0)

    def exchange_and_squash(mult):
        pltpu.sync_copy(svec, shared.at[s_ax])
        plsc.subcore_barrier()
        pltpu.sync_copy(shared.at[jnp.bitwise_xor(s_ax, 1)], pbuf)
        plsc.subcore_barrier()

        def comb_j(j, carry):
            svec[j, :] = (svec[j, :] + pbuf[j, :]) * mult
            return carry
        lax.fori_loop(0, J, comb_j, 0)

        for g in range(J // L):
            sq = zero
            for n in range(N):
                col = plsc.load_gather(svec, [iota16 + (L * g),
                                              jnp.broadcast_to(n, (L,))])
                sq = sq + col * col
            rinv = _rsqrt_vec(sq + 1e-8)
            fbuf[g, :] = (sq / (1.0 + sq)) * rinv

        def body_j(j, carry):
            f = plsc.load_gather(
                fbuf, [jnp.broadcast_to(lax.div(j, L), (L,)),
                       jnp.broadcast_to(lax.rem(j, L), (L,))])
            vvec[j, :] = svec[j, :] * f
            return carry
        lax.fori_loop(0, J, body_j, 0)

    def routing_pass(is_second):
        def ph1_slab(j, carry):
            p = lax.rem(j, 2)
            start_slab(j, p).wait()
            pl.when(j + 1 < J)(lambda: start_slab(j + 1, 1 - p).start())
            vj = vvec[j, :]
            vb = [jnp.broadcast_to(vj[n], (L,)) for n in range(N)]

            def blk(q, cq):
                for k in range(4):
                    e0, e1 = zero, zero
                    for n in range(N):
                        g = plsc.load_gather(
                            u_buf.at[p, q],
                            [idxt[16 * k + n, :], idxt[64 + 16 * k + n, :]])
                        if n % 2 == 0:
                            e0 = e0 + g * vb[n]
                        else:
                            e1 = e1 + g * vb[n]
                    off = q * 64 + 16 * k
                    bnew = (e0 + e1) * scale
                    if is_second:
                        bnew = bnew + bbuf[j, pl.ds(off, 16)]
                    bbuf[j, pl.ds(off, 16)] = bnew
                return cq
            lax.fori_loop(0, IH // 64, blk, 0)
            return carry

        start_slab(0, 0).start()
        lax.fori_loop(0, J, ph1_slab, 0)

        def ph2_k(k, ck):
            base = k * 16
            work = []
            for j in range(J):
                bj = bbuf[j, pl.ds(base, 16)]
                if is_second:
                    wj = jnp.where(bj == minf, pinf, bj)
                    wbuf[j, :] = wj
                else:
                    wj = bj
                work.append(wj)
            a_half = _sort16(work[0:16])
            b_half = _sort16(work[16:32])
            kth = _tree_max([jnp.minimum(a_half[i], b_half[7 - i])
                             for i in range(8)])

            mx = minf
            for j in range(J):
                bj = bbuf[j, pl.ds(base, 16)]
                wj = wbuf[j, :] if is_second else bj
                masked = wj <= kth
                if is_second:
                    masked = jnp.logical_or(masked, bj == minf)
                out = jnp.where(masked, minf, bj)
                if not is_second:
                    bbuf[j, pl.ds(base, 16)] = out
                mx = jnp.maximum(mx, out)

            z0, z1 = zero, zero
            for j in range(J):
                bj = bbuf[j, pl.ds(base, 16)]
                if is_second:
                    wj = wbuf[j, :]
                    masked = jnp.logical_or(wj <= kth, bj == minf)
                else:
                    masked = bj == minf
                e = jnp.where(masked, zero, jnp.exp(
                    jnp.where(masked, zero, bj - mx)))
                cbuf[j, pl.ds(base, 16)] = e
                if j % 2 == 0:
                    z0 = z0 + e
                else:
                    z1 = z1 + e
            rz = 1.0 / (z0 + z1)
            for j in range(J):
                cbuf[j, pl.ds(base, 16)] = cbuf[j, pl.ds(base, 16)] * rz
            return ck
        lax.fori_loop(0, IH // 16, ph2_k, 0)

        lax.fori_loop(0, J, zero_svec, 0)

        def ph3_slab(j, carry):
            p = lax.rem(j, 2)
            start_slab(j, p).wait()
            pl.when(j + 1 < J)(lambda: start_slab(j + 1, 1 - p).start())

            def blk(q, acc2):
                accs = [zero] * 4
                for k in range(4):
                    cv = cbuf[j, pl.ds(q * 64 + 16 * k, 16)]
                    for t in range(16):
                        cs = jnp.broadcast_to(cv[t], (L,))
                        accs[k] = accs[k] + cs * u_nat(p, q, (16 * k + t) * N)
                return acc2 + ((accs[0] + accs[1]) + (accs[2] + accs[3]))
            acc = lax.fori_loop(0, IH // 64, blk, zero)
            svec[j, :] = acc
            return carry

        start_slab(0, 0).start()
        lax.fori_loop(0, J, ph3_slab, 0)

    exchange_and_squash(scale * (1.0 / J))

    routing_pass(is_second=False)
    exchange_and_squash(scale)

    routing_pass(is_second=True)
    exchange_and_squash(scale)

    pl.when(ihalf == 0)(lambda: pltpu.sync_copy(vvec, v_out.at[b_idx]))


_routing = pl.kernel(
    _routing_body,
    out_type=jax.ShapeDtypeStruct((B, J, N), jnp.float32),
    mesh=plsc.VectorSubcoreMesh(core_axis_name="c", subcore_axis_name="s",
                                num_cores=NC, num_subcores=NS),
    compiler_params=pltpu.CompilerParams(
        needs_layout_passes=False, use_tc_tiling_on_sc=False),
    scratch_types=[
        pltpu.VMEM((2, SW // XROW, 8, 128), jnp.float32),
        pltpu.VMEM((J, IH), jnp.float32),
        pltpu.VMEM((J, IH), jnp.float32),
        pltpu.VMEM((J, L), jnp.float32),
        pltpu.VMEM((J, N), jnp.float32),
        pltpu.VMEM((J, N), jnp.float32),
        pltpu.VMEM((J, N), jnp.float32),
        pltpu.VMEM((L,), jnp.float32),
        pltpu.VMEM((2 * 4 * N, L), jnp.int32),
        pltpu.VMEM((J // L, L), jnp.float32),
        pltpu.VMEM_SHARED((NS, J, N), jnp.float32),
        pltpu.SemaphoreType.DMA((2,)),
    ],
)


def kernel(u_hat, iters):
    scale = jnp.asarray(iters, jnp.float32) / 3.0
    scale_arr = jnp.broadcast_to(scale, (L,)).astype(jnp.float32)
    u_t = u_hat.reshape(B * J * I * N // XROW, 8, 128)
    return _routing(u_t, scale_arr)


# speedup vs baseline: 1.3107x; 1.3107x over previous
import functools

import jax
import jax.numpy as jnp
from jax import lax
from jax.experimental import pallas as pl
from jax.experimental.pallas import tpu as pltpu
from jax.experimental.pallas import tpu_sc as plsc

B, J, I, N = 16, 32, 2048, 16
L = 16
NC, NS = 2, 16
IH = I // 2
IC = 64
NCH = IH // IC
CW = IC * N
NEG = float("-inf")
POS = float("inf")


def _sort16(v):
    k = 2
    while k <= 16:
        step = k // 2
        while step >= 1:
            for i in range(16):
                m = i ^ step
                if m > i:
                    a, b = v[i], v[m]
                    mn = jnp.minimum(a, b)
                    mx = jnp.maximum(a, b)
                    if (i & k) == 0:
                        v[i], v[m] = mn, mx
                    else:
                        v[i], v[m] = mx, mn
            step //= 2
        k *= 2
    return v


def _tree_max(vs):
    while len(vs) > 1:
        vs = [jnp.maximum(vs[i], vs[i + 1]) for i in range(0, len(vs) - 1, 2)] + (
            [vs[-1]] if len(vs) % 2 else [])
    return vs[0]


def _rsqrt_vec(x):
    i = plsc.bitcast(x, jnp.int32)
    y = plsc.bitcast(jnp.int32(0x5F3759DF) - (i >> 1), jnp.float32)
    for _ in range(3):
        y = y * (1.5 - 0.5 * x * y * y)
    return y


def _routing_body(u_hbm, scale_hbm, v_out, u_buf, bbuf, cbuf, wbuf,
                  svec, pbuf, vvec, scvec, idxt, fbuf, shared, dsem):
    c_ax = lax.axis_index("c")
    s_ax = lax.axis_index("s")
    b_idx = c_ax * 8 + lax.div(s_ax, 2)
    ihalf = lax.rem(s_ax, 2)
    i_base = ihalf * IH

    pltpu.sync_copy(scale_hbm, scvec)
    scale = scvec[...]
    zero = jnp.zeros((L,), jnp.float32)
    minf = jnp.full((L,), NEG, jnp.float32)
    pinf = jnp.full((L,), POS, jnp.float32)
    iota16 = lax.iota(jnp.int32, L)

    for k in range(IC // 16):
        for n in range(N):
            idxt[16 * k + n, :] = iota16 * N + (16 * N * k + n)

    def start_chunk(ci, p):
        src = u_hbm.at[b_idx, :, pl.ds(i_base * N + ci * CW, CW)]
        return pltpu.make_async_copy(src, u_buf.at[p], dsem.at[p])

    def zero_svec(j, carry):
        svec[j, :] = zero
        return carry

    lax.fori_loop(0, J, zero_svec, 0)

    def p0_chunk(ci, carry):
        p = lax.rem(ci, 2)
        start_chunk(ci, p).wait()
        pl.when(ci + 1 < NCH)(lambda: start_chunk(ci + 1, 1 - p).start())

        def body_j(j, cj):
            accs = [zero] * 4
            for k in range(IC // 16):
                for t in range(16):
                    accs[k] = accs[k] + u_buf[p, j, pl.ds((16 * k + t) * N, L)]
            plsc.addupdate(svec.at[j], (accs[0] + accs[1]) + (accs[2] + accs[3]))
            return cj
        lax.fori_loop(0, J, body_j, 0)
        return carry

    start_chunk(0, 0).start()
    lax.fori_loop(0, NCH, p0_chunk, 0)

    def exchange_and_squash(mult):
        pltpu.sync_copy(svec, shared.at[s_ax])
        plsc.subcore_barrier()
        pltpu.sync_copy(shared.at[jnp.bitwise_xor(s_ax, 1)], pbuf)
        plsc.subcore_barrier()

        def comb_j(j, carry):
            svec[j, :] = (svec[j, :] + pbuf[j, :]) * mult
            return carry
        lax.fori_loop(0, J, comb_j, 0)

        for g in range(J // L):
            sq = zero
            for n in range(N):
                col = plsc.load_gather(svec, [iota16 + (L * g),
                                              jnp.broadcast_to(n, (L,))])
                sq = sq + col * col
            rinv = _rsqrt_vec(sq + 1e-8)
            fbuf[g, :] = (sq / (1.0 + sq)) * rinv

        def body_j(j, carry):
            f = plsc.load_gather(
                fbuf, [jnp.broadcast_to(lax.div(j, L), (L,)),
                       jnp.broadcast_to(lax.rem(j, L), (L,))])
            vvec[j, :] = svec[j, :] * f
            return carry
        lax.fori_loop(0, J, body_j, 0)

    def routing_pass(is_second):
        lax.fori_loop(0, J, zero_svec, 0)

        def chunk(ci, carry):
            p = lax.rem(ci, 2)
            start_chunk(ci, p).wait()
            pl.when(ci + 1 < NCH)(lambda: start_chunk(ci + 1, 1 - p).start())
            boff = ci * IC

            def ph1_one(j):
                vj = vvec[j, :]
                vb = [jnp.broadcast_to(vj[n], (L,)) for n in range(N)]
                for k in range(IC // 16):
                    e0, e1 = zero, zero
                    for n in range(N):
                        g = plsc.load_gather(u_buf.at[p, j],
                                             [idxt[16 * k + n, :]])
                        if n % 2 == 0:
                            e0 = e0 + g * vb[n]
                        else:
                            e1 = e1 + g * vb[n]
                    bnew = (e0 + e1) * scale
                    if is_second:
                        bnew = bnew + bbuf[j, pl.ds(boff + 16 * k, 16)]
                    bbuf[j, pl.ds(boff + 16 * k, 16)] = bnew

            def ph1_j(j2, cj):
                ph1_one(j2 * 2)
                ph1_one(j2 * 2 + 1)
                return cj
            lax.fori_loop(0, J // 2, ph1_j, 0)

            def ph2_k(k, ck):
                base = boff + 16 * k
                work = []
                for j in range(J):
                    bj = bbuf[j, pl.ds(base, 16)]
                    if is_second:
                        wj = jnp.where(bj == minf, pinf, bj)
                        wbuf[j, :] = wj
                    else:
                        wj = bj
                    work.append(wj)
                a_half = _sort16(work[0:16])
                b_half = _sort16(work[16:32])
                kth = _tree_max([jnp.minimum(a_half[i], b_half[7 - i])
                                 for i in range(8)])

                mx = minf
                for j in range(J):
                    bj = bbuf[j, pl.ds(base, 16)]
                    wj = wbuf[j, :] if is_second else bj
                    masked = wj <= kth
                    if is_second:
                        masked = jnp.logical_or(masked, bj == minf)
                    out = jnp.where(masked, minf, bj)
                    if not is_second:
                        bbuf[j, pl.ds(base, 16)] = out
                    mx = jnp.maximum(mx, out)

                z0, z1 = zero, zero
                for j in range(J):
                    bj = bbuf[j, pl.ds(base, 16)]
                    if is_second:
                        wj = wbuf[j, :]
                        masked = jnp.logical_or(wj <= kth, bj == minf)
                    else:
                        masked = bj == minf
                    e = jnp.where(masked, zero, jnp.exp(
                        jnp.where(masked, zero, bj - mx)))
                    cbuf[j, pl.ds(16 * k, 16)] = e
                    if j % 2 == 0:
                        z0 = z0 + e
                    else:
                        z1 = z1 + e
                rz = 1.0 / (z0 + z1)
                for j in range(J):
                    cbuf[j, pl.ds(16 * k, 16)] = cbuf[j, pl.ds(16 * k, 16)] * rz
                return ck
            lax.fori_loop(0, IC // 16, ph2_k, 0)

            def ph3_one(j):
                accs = [zero] * 4
                for k in range(IC // 16):
                    cv = cbuf[j, pl.ds(16 * k, 16)]
                    for t in range(16):
                        cs = jnp.broadcast_to(cv[t], (L,))
                        accs[k] = accs[k] + cs * u_buf[p, j,
                                                       pl.ds((16 * k + t) * N, L)]
                plsc.addupdate(svec.at[j],
                               (accs[0] + accs[1]) + (accs[2] + accs[3]))

            def ph3_j(j2, cj):
                ph3_one(j2 * 2)
                ph3_one(j2 * 2 + 1)
                return cj
            lax.fori_loop(0, J // 2, ph3_j, 0)
            return carry

        start_chunk(0, 0).start()
        lax.fori_loop(0, NCH, chunk, 0)

    exchange_and_squash(scale * (1.0 / J))

    routing_pass(is_second=False)
    exchange_and_squash(scale)

    routing_pass(is_second=True)
    exchange_and_squash(scale)

    pl.when(ihalf == 0)(lambda: pltpu.sync_copy(vvec, v_out.at[b_idx]))


_routing = pl.kernel(
    _routing_body,
    out_type=jax.ShapeDtypeStruct((B, J, N), jnp.float32),
    mesh=plsc.VectorSubcoreMesh(core_axis_name="c", subcore_axis_name="s",
                                num_cores=NC, num_subcores=NS),
    compiler_params=pltpu.CompilerParams(
        needs_layout_passes=False, use_tc_tiling_on_sc=False),
    scratch_types=[
        pltpu.VMEM((2, J, CW), jnp.float32),
        pltpu.VMEM((J, IH), jnp.float32),
        pltpu.VMEM((J, IC), jnp.float32),
        pltpu.VMEM((J, L), jnp.float32),
        pltpu.VMEM((J, N), jnp.float32),
        pltpu.VMEM((J, N), jnp.float32),
        pltpu.VMEM((J, N), jnp.float32),
        pltpu.VMEM((L,), jnp.float32),
        pltpu.VMEM((IC, L), jnp.int32),
        pltpu.VMEM((J // L, L), jnp.float32),
        pltpu.VMEM_SHARED((NS, J, N), jnp.float32),
        pltpu.SemaphoreType.DMA((2,)),
    ],
)


def kernel(u_hat, iters):
    scale = jnp.asarray(iters, jnp.float32) / 3.0
    scale_arr = jnp.broadcast_to(scale, (L,)).astype(jnp.float32)
    u_flat = u_hat.reshape(B, J, I * N)
    return _routing(u_flat, scale_arr)


# speedup vs baseline: 1.3231x; 1.0094x over previous
import functools

import jax
import jax.numpy as jnp
from jax import lax
from jax.experimental import pallas as pl
from jax.experimental.pallas import tpu as pltpu
from jax.experimental.pallas import tpu_sc as plsc

B, J, I, N = 16, 32, 2048, 16
L = 16
NC, NS = 2, 16
IH = I // 2
IC = 64
NCH = IH // IC
CW = IC * N
NEG = float("-inf")
POS = float("inf")


def _sort16(v):
    k = 2
    while k <= 16:
        step = k // 2
        while step >= 1:
            for i in range(16):
                m = i ^ step
                if m > i:
                    a, b = v[i], v[m]
                    mn = jnp.minimum(a, b)
                    mx = jnp.maximum(a, b)
                    if (i & k) == 0:
                        v[i], v[m] = mn, mx
                    else:
                        v[i], v[m] = mx, mn
            step //= 2
        k *= 2
    return v


def _tree_max(vs):
    while len(vs) > 1:
        vs = [jnp.maximum(vs[i], vs[i + 1]) for i in range(0, len(vs) - 1, 2)] + (
            [vs[-1]] if len(vs) % 2 else [])
    return vs[0]


def _rsqrt_vec(x):
    i = plsc.bitcast(x, jnp.int32)
    y = plsc.bitcast(jnp.int32(0x5F3759DF) - (i >> 1), jnp.float32)
    for _ in range(3):
        y = y * (1.5 - 0.5 * x * y * y)
    return y


def _routing_body(u_hbm, scale_hbm, v_out, u_buf, bbuf, cbuf, wbuf,
                  svec, pbuf, vvec, scvec, idxt, fbuf, shared, dsem):
    c_ax = lax.axis_index("c")
    s_ax = lax.axis_index("s")
    b_idx = c_ax * 8 + lax.div(s_ax, 2)
    ihalf = lax.rem(s_ax, 2)
    i_base = ihalf * IH

    pltpu.sync_copy(scale_hbm, scvec)
    scale = scvec[...]
    zero = jnp.zeros((L,), jnp.float32)
    minf = jnp.full((L,), NEG, jnp.float32)
    pinf = jnp.full((L,), POS, jnp.float32)
    iota16 = lax.iota(jnp.int32, L)

    for k in range(IC // 16):
        for n in range(N):
            idxt[16 * k + n, :] = iota16 * N + (16 * N * k + n)

    def start_chunk(ci, p):
        src = u_hbm.at[b_idx, :, pl.ds(i_base * N + ci * CW, CW)]
        return pltpu.make_async_copy(src, u_buf.at[p], dsem.at[p])

    def zero_svec(j, carry):
        svec[j, :] = zero
        return carry

    lax.fori_loop(0, J, zero_svec, 0)

    def p0_chunk(ci, carry):
        p = lax.rem(ci, 2)
        start_chunk(ci, p).wait()
        pl.when(ci + 1 < NCH)(lambda: start_chunk(ci + 1, 1 - p).start())

        def body_j(j, cj):
            accs = [zero] * 4
            for k in range(IC // 16):
                for t in range(16):
                    accs[k] = accs[k] + u_buf[p, j, pl.ds((16 * k + t) * N, L)]
            plsc.addupdate(svec.at[j], (accs[0] + accs[1]) + (accs[2] + accs[3]))
            return cj
        lax.fori_loop(0, J, body_j, 0)
        return carry

    start_chunk(0, 0).start()
    lax.fori_loop(0, NCH, p0_chunk, 0)

    def exchange_and_squash(mult):
        pltpu.sync_copy(svec, shared.at[s_ax])
        plsc.subcore_barrier()
        pltpu.sync_copy(shared.at[jnp.bitwise_xor(s_ax, 1)], pbuf)
        plsc.subcore_barrier()

        def comb_j(j, carry):
            svec[j, :] = (svec[j, :] + pbuf[j, :]) * mult
            return carry
        lax.fori_loop(0, J, comb_j, 0)

        for g in range(J // L):
            sq = zero
            for n in range(N):
                col = plsc.load_gather(svec, [iota16 + (L * g),
                                              jnp.broadcast_to(n, (L,))])
                sq = sq + col * col
            rinv = _rsqrt_vec(sq + 1e-8)
            fbuf[g, :] = (sq / (1.0 + sq)) * rinv

        def body_j(j, carry):
            f = plsc.load_gather(
                fbuf, [jnp.broadcast_to(lax.div(j, L), (L,)),
                       jnp.broadcast_to(lax.rem(j, L), (L,))])
            vvec[j, :] = svec[j, :] * f
            return carry
        lax.fori_loop(0, J, body_j, 0)

    def routing_pass(is_second):
        lax.fori_loop(0, J, zero_svec, 0)

        def chunk(ci, carry):
            p = lax.rem(ci, 2)
            start_chunk(ci, p).wait()
            pl.when(ci + 1 < NCH)(lambda: start_chunk(ci + 1, 1 - p).start())
            boff = ci * IC

            def ph1_one(j):
                vj = vvec[j, :]
                vb = [jnp.broadcast_to(vj[n], (L,)) for n in range(N)]
                for k in range(IC // 16):
                    e0, e1 = zero, zero
                    for n in range(N):
                        g = plsc.load_gather(u_buf.at[p, j],
                                             [idxt[16 * k + n, :]])
                        if n % 2 == 0:
                            e0 = e0 + g * vb[n]
                        else:
                            e1 = e1 + g * vb[n]
                    bnew = (e0 + e1) * scale
                    if is_second:
                        bnew = bnew + bbuf[j, pl.ds(boff + 16 * k, 16)]
                    bbuf[j, pl.ds(boff + 16 * k, 16)] = bnew

            def ph1_j(j2, cj):
                ph1_one(j2 * 2)
                ph1_one(j2 * 2 + 1)
                return cj
            lax.fori_loop(0, J // 2, ph1_j, 0)

            def ph2_k(k, ck):
                base = boff + 16 * k
                work = []
                mxs = []
                for j in range(J):
                    bj = bbuf[j, pl.ds(base, 16)]
                    mxs.append(bj)
                    if is_second:
                        wj = jnp.where(bj == minf, pinf, bj)
                        wbuf[j, :] = wj
                    else:
                        wj = bj
                    work.append(wj)
                mx = _tree_max(mxs)
                a_half = _sort16(work[0:16])
                b_half = _sort16(work[16:32])
                kth = _tree_max([jnp.minimum(a_half[i], b_half[7 - i])
                                 for i in range(8)])

                z0, z1 = zero, zero
                for j in range(J):
                    bj = bbuf[j, pl.ds(base, 16)]
                    wj = wbuf[j, :] if is_second else bj
                    masked = wj <= kth
                    if is_second:
                        masked = jnp.logical_or(masked, bj == minf)
                    if not is_second:
                        bbuf[j, pl.ds(base, 16)] = jnp.where(masked, minf, bj)
                    e = jnp.where(masked, zero, jnp.exp(
                        jnp.where(masked, zero, bj - mx)))
                    cbuf[j, pl.ds(16 * k, 16)] = e
                    if j % 2 == 0:
                        z0 = z0 + e
                    else:
                        z1 = z1 + e
                rz = 1.0 / (z0 + z1)
                for j in range(J):
                    cbuf[j, pl.ds(16 * k, 16)] = cbuf[j, pl.ds(16 * k, 16)] * rz
                return ck
            lax.fori_loop(0, IC // 16, ph2_k, 0)

            def ph3_one(j):
                accs = [zero] * 4
                for k in range(IC // 16):
                    cv = cbuf[j, pl.ds(16 * k, 16)]
                    for t in range(16):
                        cs = jnp.broadcast_to(cv[t], (L,))
                        accs[k] = accs[k] + cs * u_buf[p, j,
                                                       pl.ds((16 * k + t) * N, L)]
                plsc.addupdate(svec.at[j],
                               (accs[0] + accs[1]) + (accs[2] + accs[3]))

            def ph3_j(j2, cj):
                ph3_one(j2 * 2)
                ph3_one(j2 * 2 + 1)
                return cj
            lax.fori_loop(0, J // 2, ph3_j, 0)
            return carry

        lax.fori_loop(0, NCH, chunk, 0)

    start_chunk(0, 0).start()
    exchange_and_squash(scale * (1.0 / J))

    routing_pass(is_second=False)
    start_chunk(0, 0).start()
    exchange_and_squash(scale)

    routing_pass(is_second=True)
    exchange_and_squash(scale)

    pl.when(ihalf == 0)(lambda: pltpu.sync_copy(vvec, v_out.at[b_idx]))


_routing = pl.kernel(
    _routing_body,
    out_type=jax.ShapeDtypeStruct((B, J, N), jnp.float32),
    mesh=plsc.VectorSubcoreMesh(core_axis_name="c", subcore_axis_name="s",
                                num_cores=NC, num_subcores=NS),
    compiler_params=pltpu.CompilerParams(
        needs_layout_passes=False, use_tc_tiling_on_sc=False),
    scratch_types=[
        pltpu.VMEM((2, J, CW), jnp.float32),
        pltpu.VMEM((J, IH), jnp.float32),
        pltpu.VMEM((J, IC), jnp.float32),
        pltpu.VMEM((J, L), jnp.float32),
        pltpu.VMEM((J, N), jnp.float32),
        pltpu.VMEM((J, N), jnp.float32),
        pltpu.VMEM((J, N), jnp.float32),
        pltpu.VMEM((L,), jnp.float32),
        pltpu.VMEM((IC, L), jnp.int32),
        pltpu.VMEM((J // L, L), jnp.float32),
        pltpu.VMEM_SHARED((NS, J, N), jnp.float32),
        pltpu.SemaphoreType.DMA((2,)),
    ],
)


def kernel(u_hat, iters):
    scale = jnp.asarray(iters, jnp.float32) / 3.0
    scale_arr = jnp.broadcast_to(scale, (L,)).astype(jnp.float32)
    u_flat = u_hat.reshape(B, J, I * N)
    return _routing(u_flat, scale_arr)
